# baseline (speedup 1.0000x reference)
import jax
import jax.numpy as jnp
from jax import lax
from jax.experimental import pallas as pl
from jax.experimental.pallas import tpu as pltpu

K_BINS = 1024
EMB = 64
TN = 2048


def _vq_body(z_ref, cb_ref, xq_ref, xl_ref, fit_ref):
    i = pl.program_id(0)
    x = z_ref[...]
    cb = cb_ref[...]
    xsq = jnp.sum(x * x, axis=1, keepdims=True)
    ksq = jnp.sum(cb * cb, axis=1)[None, :]
    xk2 = lax.dot_general(x, cb * (-2.0), (((1,), (1,)), ((), ())),
                          preferred_element_type=jnp.float32)
    dist = (xsq + xk2) + ksq
    mind = jnp.min(dist, axis=1)
    iota = lax.broadcasted_iota(jnp.int32, dist.shape, 1)
    sel = jnp.where(dist <= mind[:, None], iota, K_BINS)
    idx = jnp.min(sel, axis=1)
    onehot = (iota == idx[:, None]).astype(jnp.float32)
    xd = lax.dot_general(onehot, cb, (((1,), (0,)), ((), ())),
                         preferred_element_type=jnp.float32)
    xq_ref[...] = x + (xd - x)
    xl_ref[...] = idx

    @pl.when(i == 0)
    def _init():
        fit_ref[...] = jnp.zeros((1, 1), jnp.float32)

    fit_ref[...] += jnp.sum(mind).reshape(1, 1)


@jax.jit
def kernel(z, codebook):
    B, T, D = z.shape
    N = B * T
    x = z.reshape(N, D)
    xq, xl, fit_s = pl.pallas_call(
        _vq_body,
        grid=(N // TN,),
        in_specs=[
            pl.BlockSpec((TN, D), lambda i: (i, 0)),
            pl.BlockSpec((K_BINS, D), lambda i: (0, 0)),
        ],
        out_specs=[
            pl.BlockSpec((TN, D), lambda i: (i, 0)),
            pl.BlockSpec((TN,), lambda i: (i,)),
            pl.BlockSpec((1, 1), lambda i: (0, 0)),
        ],
        out_shape=[
            jax.ShapeDtypeStruct((N, D), jnp.float32),
            jax.ShapeDtypeStruct((N,), jnp.int32),
            jax.ShapeDtypeStruct((1, 1), jnp.float32),
        ],
    )(x, codebook)
    fit = fit_s[0, 0] / N
    commit_loss = fit / D
    return xq.reshape(B, T, D), commit_loss, fit, xl.reshape(B, T)


# speedup vs baseline: 1.1282x; 1.1282x over previous
import jax
import jax.numpy as jnp
from jax import lax
from jax.experimental import pallas as pl
from jax.experimental.pallas import tpu as pltpu

K_BINS = 1024
EMB = 64
TN = 4096


def _vq_body(z_ref, cb_ref, aug_ref, xq_ref, xl_ref, fit_ref):
    i = pl.program_id(0)
    x = z_ref[...]
    cb = cb_ref[...]
    xsq = jnp.sum(x * x, axis=1, keepdims=True)
    ksq = jnp.sum(cb * cb, axis=1)[None, :]
    xk2 = lax.dot_general(x, cb * (-2.0), (((1,), (1,)), ((), ())),
                          preferred_element_type=jnp.float32)
    dist = (xsq + xk2) + ksq
    mind = jnp.min(dist, axis=1)
    mask = (dist <= mind[:, None]).astype(jnp.float32)
    g = lax.dot_general(mask, aug_ref[...], (((1,), (0,)), ((), ())),
                        preferred_element_type=jnp.float32)
    xd = g[:, :EMB]
    cnt = g[:, EMB]
    idx = g[:, EMB + 1].astype(jnp.int32)
    xq_ref[...] = x + (xd - x)
    xl_ref[...] = idx

    @pl.when(jnp.max(cnt) > 1.0)
    def _ties():
        iota = lax.broadcasted_iota(jnp.int32, dist.shape, 1)
        sel = jnp.where(dist <= mind[:, None], iota, K_BINS)
        idx2 = jnp.min(sel, axis=1)
        onehot = (iota == idx2[:, None]).astype(jnp.float32)
        xd2 = lax.dot_general(onehot, cb, (((1,), (0,)), ((), ())),
                              preferred_element_type=jnp.float32)
        xq_ref[...] = x + (xd2 - x)
        xl_ref[...] = idx2

    @pl.when(i == 0)
    def _init():
        fit_ref[...] = jnp.zeros((1, 1), jnp.float32)

    fit_ref[...] += jnp.sum(mind).reshape(1, 1)


@jax.jit
def kernel(z, codebook):
    B, T, D = z.shape
    N = B * T
    x = z.reshape(N, D)
    iota_k = lax.iota(jnp.float32, K_BINS)[:, None]
    aug = jnp.concatenate(
        [codebook, jnp.ones((K_BINS, 1), jnp.float32), iota_k,
         jnp.zeros((K_BINS, 128 - D - 2), jnp.float32)], axis=1)
    xq, xl, fit_s = pl.pallas_call(
        _vq_body,
        grid=(N // TN,),
        in_specs=[
            pl.BlockSpec((TN, D), lambda i: (i, 0)),
            pl.BlockSpec((K_BINS, D), lambda i: (0, 0)),
            pl.BlockSpec((K_BINS, 128), lambda i: (0, 0)),
        ],
        out_specs=[
            pl.BlockSpec((TN, D), lambda i: (i, 0)),
            pl.BlockSpec((TN,), lambda i: (i,)),
            pl.BlockSpec((1, 1), lambda i: (0, 0)),
        ],
        out_shape=[
            jax.ShapeDtypeStruct((N, D), jnp.float32),
            jax.ShapeDtypeStruct((N,), jnp.int32),
            jax.ShapeDtypeStruct((1, 1), jnp.float32),
        ],
    )(x, codebook, aug)
    fit = fit_s[0, 0] / N
    commit_loss = fit / D
    return xq.reshape(B, T, D), commit_loss, fit, xl.reshape(B, T)
